# baseline (speedup 1.0000x reference)
import functools

import jax
import jax.numpy as jnp
from jax import lax
from jax.experimental import pallas as pl
from jax.experimental.pallas import tpu as pltpu
from jax.experimental.pallas import tpu_sc as plsc

B = 4
T = 64
BT = B * T
P_HEMI = 320
P2 = 2 * P_HEMI
V = 153
H = 40962
VT = 2 * H
ROWS_H = V * P_HEMI

NUM_TILES = 32
CHUNK = 120
NCHUNKS = ROWS_H // CHUNK
LANES = 16

_VB = 9
_CH_PER_VB = _VB * P_HEMI // CHUNK
_PART_BLOCKS = (2, 4, 4, 4, 3)

_HB = 4096
_RB0 = 10
_ROFF = _RB0 * _HB

_mesh = plsc.VectorSubcoreMesh(core_axis_name="c", subcore_axis_name="s")


def _make_sc_gather(c0, n_chunks):
    @functools.partial(
        pl.kernel,
        out_type=jax.ShapeDtypeStruct((n_chunks * CHUNK, BT), jnp.float32),
        mesh=_mesh,
        scratch_types=[
            pltpu.VMEM((CHUNK,), jnp.int32),
            pltpu.VMEM((CHUNK,), jnp.int32),
            pltpu.VMEM((CHUNK, BT), jnp.float32),
            pltpu.VMEM((CHUNK, BT), jnp.float32),
            pltpu.SemaphoreType.DMA,
            pltpu.SemaphoreType.DMA,
            pltpu.SemaphoreType.DMA,
            pltpu.SemaphoreType.DMA,
        ],
    )
    def sc_gather(xg_hbm, pi_hbm, out_hbm, idx0, idx1, buf0, buf1,
                  semg0, semg1, semo0, semo1):
        wid = lax.axis_index("s") * 2 + lax.axis_index("c")
        nchunks_w = jnp.where(wid < n_chunks % NUM_TILES,
                              n_chunks // NUM_TILES + 1,
                              n_chunks // NUM_TILES)
        idxs = (idx0, idx1)
        bufs = (buf0, buf1)
        semgs = (semg0, semg1)
        semos = (semo0, semo1)

        def load_idx(i, idx_v):
            c = wid + i * NUM_TILES
            pltpu.sync_copy(pi_hbm.at[pl.ds((c0 + c) * CHUNK, CHUNK)], idx_v)

        @pl.when(nchunks_w > 0)
        def _():
            load_idx(0, idx0)
            pltpu.async_copy(xg_hbm.at[idx0], buf0, semg0)

        def chunk_body(i, carry):
            c = wid + i * NUM_TILES
            for par in range(2):
                @pl.when(i % 2 == par)
                def _():
                    @pl.when(i + 1 < nchunks_w)
                    def _():
                        load_idx(i + 1, idxs[1 - par])

                    @pl.when(i >= 1)
                    def _():
                        pltpu.make_async_copy(
                            bufs[1 - par], out_hbm.at[pl.ds(0, CHUNK)],
                            semos[1 - par]).wait()

                    pltpu.make_async_copy(
                        xg_hbm.at[idxs[par]], bufs[par], semgs[par]).wait()

                    @pl.when(i + 1 < nchunks_w)
                    def _():
                        pltpu.async_copy(
                            xg_hbm.at[idxs[1 - par]], bufs[1 - par],
                            semgs[1 - par])

                    pltpu.async_copy(bufs[par],
                                     out_hbm.at[pl.ds(c * CHUNK, CHUNK)],
                                     semos[par])
            return carry

        lax.fori_loop(0, nchunks_w, chunk_body, 0)

        @pl.when(nchunks_w > 0)
        def _():
            for par in range(2):
                @pl.when((nchunks_w - 1) % 2 == par)
                def _():
                    pltpu.make_async_copy(bufs[par],
                                          out_hbm.at[pl.ds(0, CHUNK)],
                                          semos[par]).wait()

    return sc_gather


def _part_offsets():
    offs = []
    o = 0
    for nb in _PART_BLOCKS:
        offs.append(o)
        o += nb
    return tuple(offs)


_PART_OFFS = _part_offsets()
_sc_gather_l = _make_sc_gather(0, NCHUNKS)
_sc_gathers_r = tuple(
    _make_sc_gather(off * _CH_PER_VB, nb * _CH_PER_VB)
    for off, nb in zip(_PART_OFFS, _PART_BLOCKS))


def _repack_body(in_ref, out_ref):
    out_ref[...] = jnp.concatenate([in_ref[b].T for b in range(B)], axis=1)


def _stage_a(xt, b0, nrows):
    return pl.pallas_call(
        _repack_body,
        grid=(pl.cdiv(nrows, _HB),),
        in_specs=[pl.BlockSpec((B, T, _HB), lambda h: (0, 0, h + b0))],
        out_specs=pl.BlockSpec((_HB, BT), lambda h: (h, 0)),
        out_shape=jax.ShapeDtypeStruct((nrows, BT), jnp.float32),
    )(xt)


def _unpack_body(gl_ref, gr_ref, out_ref):
    dl = gl_ref[...].T
    dr = gr_ref[...].T
    for b in range(B):
        for vi in range(_VB):
            out_ref[b, vi] = jnp.concatenate(
                [dl[b * T:(b + 1) * T, vi * P_HEMI:(vi + 1) * P_HEMI],
                 dr[b * T:(b + 1) * T, vi * P_HEMI:(vi + 1) * P_HEMI]],
                axis=1)


def _unpack_body2(gl_ref, gr_ref, _prev_ref, out_ref):
    _unpack_body(gl_ref, gr_ref, out_ref)


def _stage_c(gl, grs):
    out_shape = jax.ShapeDtypeStruct((B, V, T, P2), jnp.float32)
    blk = _VB * P_HEMI
    out = pl.pallas_call(
        _unpack_body,
        grid=(_PART_BLOCKS[0],),
        in_specs=[
            pl.BlockSpec((blk, BT), lambda v: (v, 0)),
            pl.BlockSpec((blk, BT), lambda v: (v, 0)),
        ],
        out_specs=pl.BlockSpec((B, _VB, T, P2), lambda v: (0, v, 0, 0)),
        out_shape=out_shape,
    )(gl, grs[0])
    for gr, off, nb in zip(grs[1:], _PART_OFFS[1:], _PART_BLOCKS[1:]):
        out = pl.pallas_call(
            _unpack_body2,
            grid=(nb,),
            in_specs=[
                pl.BlockSpec((blk, BT), lambda v, off=off: (v + off, 0)),
                pl.BlockSpec((blk, BT), lambda v: (v, 0)),
                pl.BlockSpec(memory_space=pl.ANY),
            ],
            out_specs=pl.BlockSpec((B, _VB, T, P2),
                                   lambda v, off=off: (0, v + off, 0, 0)),
            out_shape=out_shape,
            input_output_aliases={2: 0},
        )(gl, gr, out)
    return out


def kernel(x, patch_indices):
    xt = jnp.transpose(x, (0, 2, 1))
    piT = patch_indices.T
    pi_l = piT.reshape(ROWS_H)
    pi_r = pi_l + (H - _ROFF)
    xg_l = _stage_a(xt, 0, (_RB0 + 1) * _HB)
    xg_r = _stage_a(xt, _RB0, VT - _ROFF)
    g_l = _sc_gather_l(xg_l, pi_l)
    g_rs = [sc(xg_r, pi_r) for sc in _sc_gathers_r]
    out_perm = _stage_c(g_l, g_rs)
    return jnp.transpose(out_perm, (0, 2, 3, 1))


# speedup vs baseline: 1.0621x; 1.0621x over previous
import functools

import jax
import jax.numpy as jnp
from jax import lax
from jax.experimental import pallas as pl
from jax.experimental.pallas import tpu as pltpu
from jax.experimental.pallas import tpu_sc as plsc

B = 4
T = 64
BT = B * T
P_HEMI = 320
P2 = 2 * P_HEMI
V = 153
H = 40962
VT = 2 * H
ROWS_TOTAL = V * P2

NUM_TILES = 32
CHUNK = 128
NCHUNKS = ROWS_TOTAL // CHUNK
LANES = 16

_VB = 9
_V_SPLIT = 72
_CH_SPLIT = _V_SPLIT * P2 // CHUNK

_mesh = plsc.VectorSubcoreMesh(core_axis_name="c", subcore_axis_name="s")


def _make_sc_gather(c0, n_chunks):
    @functools.partial(
        pl.kernel,
        out_type=jax.ShapeDtypeStruct((n_chunks * CHUNK, BT), jnp.float32),
        mesh=_mesh,
        scratch_types=[
            pltpu.VMEM((CHUNK,), jnp.int32),
            pltpu.VMEM((CHUNK,), jnp.int32),
            pltpu.VMEM((CHUNK, BT), jnp.float32),
            pltpu.VMEM((CHUNK, BT), jnp.float32),
            pltpu.SemaphoreType.DMA,
            pltpu.SemaphoreType.DMA,
            pltpu.SemaphoreType.DMA,
            pltpu.SemaphoreType.DMA,
        ],
    )
    def sc_gather(xg_hbm, pi2_hbm, out_hbm, idx0, idx1, buf0, buf1,
                  semg0, semg1, semo0, semo1):
        wid = lax.axis_index("s") * 2 + lax.axis_index("c")
        nchunks_w = jnp.where(wid < n_chunks % NUM_TILES,
                              n_chunks // NUM_TILES + 1,
                              n_chunks // NUM_TILES)
        idxs = (idx0, idx1)
        bufs = (buf0, buf1)
        semgs = (semg0, semg1)
        semos = (semo0, semo1)

        def load_idx(i, idx_v):
            c = wid + i * NUM_TILES
            pltpu.sync_copy(pi2_hbm.at[pl.ds((c0 + c) * CHUNK, CHUNK)], idx_v)

        load_idx(0, idx0)
        pltpu.async_copy(xg_hbm.at[idx0], buf0, semg0)

        def chunk_body(i, carry):
            c = wid + i * NUM_TILES
            for par in range(2):
                @pl.when(i % 2 == par)
                def _():
                    @pl.when(i + 1 < nchunks_w)
                    def _():
                        load_idx(i + 1, idxs[1 - par])

                    @pl.when(i >= 1)
                    def _():
                        pltpu.make_async_copy(
                            bufs[1 - par], out_hbm.at[pl.ds(0, CHUNK)],
                            semos[1 - par]).wait()

                    pltpu.make_async_copy(
                        xg_hbm.at[idxs[par]], bufs[par], semgs[par]).wait()

                    @pl.when(i + 1 < nchunks_w)
                    def _():
                        pltpu.async_copy(
                            xg_hbm.at[idxs[1 - par]], bufs[1 - par],
                            semgs[1 - par])

                    pltpu.async_copy(bufs[par],
                                     out_hbm.at[pl.ds(c * CHUNK, CHUNK)],
                                     semos[par])
            return carry

        lax.fori_loop(0, nchunks_w, chunk_body, 0)

        for par in range(2):
            @pl.when((nchunks_w - 1) % 2 == par)
            def _():
                pltpu.make_async_copy(bufs[par], out_hbm.at[pl.ds(0, CHUNK)],
                                      semos[par]).wait()

    return sc_gather


_sc_gather_a = _make_sc_gather(0, _CH_SPLIT)
_sc_gather_b = _make_sc_gather(_CH_SPLIT, NCHUNKS - _CH_SPLIT)


_HB = 4096


def _repack_body(in_ref, out_ref):
    out_ref[...] = jnp.concatenate([in_ref[b].T for b in range(B)], axis=1)


def _stage_a(xt):
    return pl.pallas_call(
        _repack_body,
        grid=(pl.cdiv(VT, _HB),),
        in_specs=[pl.BlockSpec((B, T, _HB), lambda h: (0, 0, h))],
        out_specs=pl.BlockSpec((_HB, BT), lambda h: (h, 0)),
        out_shape=jax.ShapeDtypeStruct((VT, BT), jnp.float32),
    )(xt)


def _unpack_body(g_ref, out_ref):
    data_t = g_ref[...].T
    for b in range(B):
        for vi in range(_VB):
            out_ref[b, vi] = data_t[b * T:(b + 1) * T,
                                    vi * P2:(vi + 1) * P2]


def _unpack_body2(g_ref, _prev_ref, out_ref):
    _unpack_body(g_ref, out_ref)


def _stage_c(g_a, g_b):
    out_shape = jax.ShapeDtypeStruct((B, V, T, P2), jnp.float32)
    part1 = pl.pallas_call(
        _unpack_body,
        grid=(_V_SPLIT // _VB,),
        in_specs=[pl.BlockSpec((_VB * P2, BT), lambda v: (v, 0))],
        out_specs=pl.BlockSpec((B, _VB, T, P2), lambda v: (0, v, 0, 0)),
        out_shape=out_shape,
    )(g_a)
    return pl.pallas_call(
        _unpack_body2,
        grid=((V - _V_SPLIT) // _VB,),
        in_specs=[
            pl.BlockSpec((_VB * P2, BT), lambda v: (v, 0)),
            pl.BlockSpec(memory_space=pl.ANY),
        ],
        out_specs=pl.BlockSpec((B, _VB, T, P2),
                               lambda v: (0, v + _V_SPLIT // _VB, 0, 0)),
        out_shape=out_shape,
        input_output_aliases={1: 0},
    )(g_b, part1)


def kernel(x, patch_indices):
    xt = jnp.transpose(x, (0, 2, 1))
    piT = patch_indices.T
    pi2 = jnp.concatenate([piT, piT + H], axis=1).reshape(ROWS_TOTAL)
    xg = _stage_a(xt)
    g_a = _sc_gather_a(xg, pi2)
    g_b = _sc_gather_b(xg, pi2)
    out_perm = _stage_c(g_a, g_b)
    return jnp.transpose(out_perm, (0, 2, 3, 1))
